# Initial kernel scaffold; baseline (speedup 1.0000x reference)
import jax
import jax.numpy as jnp
from jax.experimental import pallas as pl


def kernel(x, logits):
    raise NotImplementedError("write your pallas kernel here")


# speedup vs baseline: 1.1887x; 1.1887x over previous
import functools

import jax
import jax.numpy as jnp
from jax import lax
from jax.experimental import pallas as pl
from jax.experimental.pallas import tpu as pltpu
from jax.experimental.pallas import tpu_sc as plsc

B = 16384
V = 10
D = 784
NC = 2
NS = 16
NW = NC * NS
BPW = B // NW
C = 64
NCHUNK = BPW // C


def _make_sc_gather():
    mesh = plsc.VectorSubcoreMesh(core_axis_name="c", subcore_axis_name="s")

    @functools.partial(
        pl.kernel,
        mesh=mesh,
        compiler_params=pltpu.CompilerParams(use_tc_tiling_on_sc=False),
        out_type=jax.ShapeDtypeStruct((B, D), jnp.float32),
        scratch_types=[
            pltpu.VMEM((BPW,), jnp.int32),
            pltpu.VMEM((2, C, D), jnp.float32),
            pltpu.SemaphoreType.DMA,
            pltpu.SemaphoreType.DMA,
            pltpu.SemaphoreType.DMA,
            pltpu.SemaphoreType.DMA,
        ],
    )
    def k(idx_hbm, table_hbm, out_hbm, idx_v, buf_v, gsem0, gsem1, ssem0, ssem1):
        wid = lax.axis_index("s") * NC + lax.axis_index("c")
        base = wid * BPW
        pltpu.sync_copy(idx_hbm.at[pl.ds(base, BPW)], idx_v)

        gsems = (gsem0, gsem1)
        ssems = (ssem0, ssem1)
        gathers = [None, None]
        outs = [None, None]
        gathers[0] = pltpu.async_copy(
            table_hbm.at[idx_v.at[pl.ds(0, C)]], buf_v.at[0], gsems[0])
        for kk in range(NCHUNK):
            cur = kk % 2
            nxt = (kk + 1) % 2
            gathers[cur].wait()
            if kk + 1 < NCHUNK:
                if outs[nxt] is not None:
                    outs[nxt].wait()
                gathers[nxt] = pltpu.async_copy(
                    table_hbm.at[idx_v.at[pl.ds((kk + 1) * C, C)]],
                    buf_v.at[nxt], gsems[nxt])
            outs[cur] = pltpu.async_copy(
                buf_v.at[cur], out_hbm.at[pl.ds(base + kk * C, C)], ssems[cur])
        outs[0].wait()
        outs[1].wait()

    return k


_sc_gather = _make_sc_gather()


def kernel(x, logits):
    table = logits.reshape(V, D)
    out = _sc_gather(x.astype(jnp.int32), table)
    return out.reshape(B, 1, 28, 28)
